# baseline (speedup 1.0000x reference)
import functools

import jax
import jax.numpy as jnp
import numpy as np
from jax import lax
from jax.experimental import pallas as pl
from jax.experimental.pallas import tpu as pltpu
from jax.experimental.pallas import tpu_sc as plsc

N = 10000
NPAD = 10240
D = 64
B = 4
K = 20
KP = 24
BR = 256
NBLK = NPAD // BR
XA_W = 272
C_BN = float(1.0 / np.sqrt(1.0 + 1e-5))
NEG = float(-3.0e38)


def _topk_body(nw_all_ref, nw_blk_ref, out_ref):
    s = jax.lax.dot_general(nw_blk_ref[...], nw_all_ref[...],
                            (((1,), (1,)), ((), ())),
                            preferred_element_type=jnp.float32)
    col = jax.lax.broadcasted_iota(jnp.int32, (BR, NPAD), 1)
    s = jnp.where(col < N, s, NEG)
    colf = jax.lax.broadcasted_iota(jnp.int32, (NPAD, 1), 0).astype(jnp.float32)
    m = jnp.max(s, axis=1, keepdims=True)
    pid = pl.program_id(0)
    rows = pid * BR + jax.lax.broadcasted_iota(jnp.int32, (BR, KP), 0)
    kcol = jax.lax.broadcasted_iota(jnp.int32, (BR, KP), 1)

    def body(t, carry):
        s, m, outacc = carry
        eq = s == m
        eqf = jnp.where(eq, 1.0, 0.0)
        self_f = jax.lax.dot_general(eqf, colf, (((1,), (0,)), ((), ())),
                                     precision=jax.lax.Precision.HIGHEST,
                                     preferred_element_type=jnp.float32)
        sel = jnp.minimum(self_f, float(NPAD - 1)).astype(jnp.int32)
        outacc = jnp.where(kcol == t, sel, outacc)
        s = jnp.where(eq, NEG, s)
        m = jnp.max(s, axis=1, keepdims=True)
        return (s, m, outacc)

    _, _, outacc = jax.lax.fori_loop(0, K, body, (s, m, rows))
    out_ref[...] = outacc


def _prep_body(datap_ref, embp_ref, linW_ref, ai_ref, aj_ref, aei_ref,
               aej_ref, xa_ref):
    e = embp_ref[...]
    lw = linW_ref[...]
    pieces = []
    asrc, adst = [], []
    e_src = jnp.sum(e * aej_ref[...], axis=1, keepdims=True)
    e_dst = jnp.sum(e * aei_ref[...], axis=1, keepdims=True)
    for b in range(B):
        xl = jax.lax.dot_general(datap_ref[b], lw,
                                 (((1,), (1,)), ((), ())),
                                 preferred_element_type=jnp.float32)
        pieces.append(xl)
        asrc.append(jnp.sum(xl * aj_ref[...], axis=1, keepdims=True) + e_src)
        adst.append(jnp.sum(xl * ai_ref[...], axis=1, keepdims=True) + e_dst)
    zeros = jnp.zeros((BR, 8), jnp.float32)
    xa_ref[...] = jnp.concatenate(pieces + asrc + adst + [zeros], axis=1)


def _post_body(agg_ref, embp_ref, bias_ref, bn1g_ref, bn1b_ref, bog_ref,
               bob_ref, ow_ref, ob_ref, y_ref):
    e = embp_ref[...]
    rows = []
    for b in range(B):
        agg = agg_ref[:, b * D:(b + 1) * D] + bias_ref[...]
        o = jnp.maximum(bn1g_ref[...] * agg * C_BN + bn1b_ref[...], 0.0)
        h = o * e
        h = jnp.maximum(bog_ref[...] * h * C_BN + bob_ref[...], 0.0)
        yb = jnp.sum(h * ow_ref[...], axis=1) + ob_ref[0, 0]
        rows.append(yb[None, :])
    y_ref[...] = jnp.concatenate(rows, axis=0)


NTILE = 32
NPT = NPAD // NTILE
GRP = 8
NGRP = NPT // GRP
ROWS_G = GRP * KP


def _sc_mp_body(idx_hbm, xa_hbm, out_hbm, idxbuf, rows, accg, sem):
    wid = lax.axis_index("s") * 2 + lax.axis_index("c")
    io = lax.iota(jnp.int32, 16)
    fz = jnp.zeros((16,), jnp.float32)

    def group_body(g, _):
        base = wid * NPT + g * GRP
        pltpu.sync_copy(idx_hbm.at[pl.ds(base * KP, 96)], idxbuf.at[0])
        pltpu.sync_copy(idx_hbm.at[pl.ds(base * KP + 96, 96)], idxbuf.at[1])
        c0 = pltpu.async_copy(xa_hbm.at[idxbuf.at[0]],
                              rows.at[pl.ds(0, 96)], sem)
        c1 = pltpu.async_copy(xa_hbm.at[idxbuf.at[1]],
                              rows.at[pl.ds(96, 96)], sem)
        c0.wait()
        c1.wait()

        def node_body(n, _n):
            i = base + n
            sb = n * KP
            r = n // 4
            off = (n % 4) * KP
            iv0 = idxbuf[r, pl.ds(off, 16)]
            iv1 = idxbuf[r, pl.ds(off + 8, 16)]
            valid0 = iv0 != i
            valid1 = ((iv1 != i) | (io == 12)) & (io >= 8)
            w0, w1 = [], []
            for b in range(B):
                asrc0 = plsc.load_gather(
                    rows, [sb + io, jnp.full((16,), 4 * D + b, jnp.int32)])
                asrc1 = plsc.load_gather(
                    rows, [sb + 8 + io, jnp.full((16,), 4 * D + b, jnp.int32)])
                adv = plsc.load_gather(
                    rows, [jnp.full((16,), sb + K, jnp.int32),
                           jnp.full((16,), 4 * D + B + b, jnp.int32)])
                a0 = adv + asrc0
                a0 = jnp.where(a0 >= 0, a0, 0.2 * a0)
                a1 = adv + asrc1
                a1 = jnp.where(a1 >= 0, a1, 0.2 * a1)
                m = jnp.maximum(jnp.max(jnp.where(valid0, a0, NEG)),
                                jnp.max(jnp.where(valid1, a1, NEG)))
                e0 = jnp.where(valid0, jnp.exp(a0 - m), fz)
                e1 = jnp.where(valid1, jnp.exp(a1 - m), fz)
                den = jnp.broadcast_to(jnp.sum(e0) + jnp.sum(e1) + 1e-16,
                                       (16,))
                w0.append(e0 / den)
                w1.append(e1 / den)
            acc = [fz] * 16
            dn = lax.GatherDimensionNumbers(offset_dims=(),
                                            collapsed_slice_dims=(0,),
                                            start_index_map=(0,))
            for t in range(K + 1):
                lane = jnp.full((16, 1), t if t < 16 else t - 8, jnp.int32)
                wts = [lax.gather(w0[b] if t < 16 else w1[b], lane, dn, (1,),
                                  mode=lax.GatherScatterMode.PROMISE_IN_BOUNDS)
                       for b in range(B)]
                for c in range(16):
                    acc[c] = acc[c] + wts[c // 4] * rows[sb + t,
                                                         pl.ds(c * 16, 16)]
            for c in range(16):
                accg[n, pl.ds(c * 16, 16)] = acc[c]
            return _n

        lax.fori_loop(0, GRP, node_body, 0)
        pltpu.sync_copy(accg, out_hbm.at[pl.ds(base, GRP)])
        return _

    lax.fori_loop(0, NGRP, group_body, 0)


def _sc_mp(idx_flat, xa):
    mesh = plsc.VectorSubcoreMesh(core_axis_name="c", subcore_axis_name="s")
    return pl.kernel(
        _sc_mp_body,
        mesh=mesh,
        compiler_params=pltpu.CompilerParams(use_tc_tiling_on_sc=False,
                                             needs_layout_passes=False),
        out_type=jax.ShapeDtypeStruct((NPAD, B * D), jnp.float32),
        scratch_types=[
            pltpu.VMEM((2, 96), jnp.int32),
            pltpu.VMEM((ROWS_G, XA_W), jnp.float32),
            pltpu.VMEM((GRP, B * D), jnp.float32),
            pltpu.SemaphoreType.DMA,
        ],
    )(idx_flat, xa)


def _post(agg_all, embp, gnn_bias, bn1_gamma, bn1_beta, bn_out_gamma,
          bn_out_beta, out_W, out_b):
    yp = pl.pallas_call(
        _post_body,
        grid=(NBLK,),
        in_specs=[
            pl.BlockSpec((BR, B * D), lambda i: (i, 0)),
            pl.BlockSpec((BR, D), lambda i: (i, 0)),
            pl.BlockSpec((1, D), lambda i: (0, 0)),
            pl.BlockSpec((1, D), lambda i: (0, 0)),
            pl.BlockSpec((1, D), lambda i: (0, 0)),
            pl.BlockSpec((1, D), lambda i: (0, 0)),
            pl.BlockSpec((1, D), lambda i: (0, 0)),
            pl.BlockSpec((1, D), lambda i: (0, 0)),
            pl.BlockSpec((1, 1), lambda i: (0, 0)),
        ],
        out_specs=pl.BlockSpec((B, BR), lambda i: (0, i)),
        out_shape=jax.ShapeDtypeStruct((B, NPAD), jnp.float32),
    )(agg_all, embp, gnn_bias[None, :], bn1_gamma[None, :],
      bn1_beta[None, :], bn_out_gamma[None, :], bn_out_beta[None, :],
      out_W, out_b[:, None])
    return yp[:, :N]


def kernel(data, emb, lin_W, att_i, att_j, att_em_i, att_em_j, gnn_bias,
           bn1_gamma, bn1_beta, bn_out_gamma, bn_out_beta, out_W, out_b):
    f32 = jnp.float32
    embp = jnp.pad(emb, ((0, NPAD - N), (0, 0)))
    nw = embp / jnp.sqrt(jnp.sum(embp * embp, axis=1, keepdims=True))
    nw = jnp.where(jnp.isfinite(nw), nw, 0.0)

    topk = pl.pallas_call(
        _topk_body,
        grid=(NBLK,),
        in_specs=[
            pl.BlockSpec((NPAD, D), lambda i: (0, 0)),
            pl.BlockSpec((BR, D), lambda i: (i, 0)),
        ],
        out_specs=pl.BlockSpec((BR, KP), lambda i: (i, 0)),
        out_shape=jax.ShapeDtypeStruct((NPAD, KP), jnp.int32),
    )(nw, nw)

    datap = jnp.pad(data, ((0, 0), (0, NPAD - N), (0, 0)))
    xa = pl.pallas_call(
        _prep_body,
        grid=(NBLK,),
        in_specs=[
            pl.BlockSpec((B, BR, data.shape[-1]), lambda i: (0, i, 0)),
            pl.BlockSpec((BR, D), lambda i: (i, 0)),
            pl.BlockSpec((D, data.shape[-1]), lambda i: (0, 0)),
            pl.BlockSpec((1, D), lambda i: (0, 0)),
            pl.BlockSpec((1, D), lambda i: (0, 0)),
            pl.BlockSpec((1, D), lambda i: (0, 0)),
            pl.BlockSpec((1, D), lambda i: (0, 0)),
        ],
        out_specs=pl.BlockSpec((BR, XA_W), lambda i: (i, 0)),
        out_shape=jax.ShapeDtypeStruct((NPAD, XA_W), f32),
    )(datap, embp, lin_W, att_i[0], att_j[0], att_em_i[0], att_em_j[0])

    agg_all = _sc_mp(topk.reshape(-1), xa)
    return _post(agg_all, embp, gnn_bias, bn1_gamma, bn1_beta,
                 bn_out_gamma, bn_out_beta, out_W, out_b)


# speedup vs baseline: 3.8019x; 3.8019x over previous
import functools

import jax
import jax.numpy as jnp
import numpy as np
from jax import lax
from jax.experimental import pallas as pl
from jax.experimental.pallas import tpu as pltpu
from jax.experimental.pallas import tpu_sc as plsc

N = 10000
NPAD = 10240
D = 64
B = 4
K = 20
KP = 24
BR = 512
NBLK = NPAD // BR
XA_W = 272
C_BN = float(1.0 / np.sqrt(1.0 + 1e-5))
NEG = float(-3.0e38)


def _topk_body(nw_all_ref, nw_blk_ref, out_ref):
    s = jax.lax.dot_general(nw_blk_ref[...], nw_all_ref[...],
                            (((1,), (1,)), ((), ())),
                            preferred_element_type=jnp.float32)
    col = jax.lax.broadcasted_iota(jnp.int32, (BR, NPAD), 1)
    s = jnp.where(col < N, s, NEG)
    sels = []
    m = jnp.max(s, axis=1, keepdims=True)
    for t in range(K):
        eq = s == m
        cand = jnp.where(eq, col, NPAD)
        sel = jnp.min(cand, axis=1, keepdims=True)
        s = jnp.where(eq, NEG, s)
        if t < K - 1:
            m = jnp.max(s, axis=1, keepdims=True)
        sels.append(sel)
    pid = pl.program_id(0)
    row = pid * BR + jax.lax.broadcasted_iota(jnp.int32, (BR, 1), 0)
    out_ref[...] = jnp.concatenate(sels + [row] * (KP - K), axis=1)


def _prep_body(datap_ref, embp_ref, linW_ref, ai_ref, aj_ref, aei_ref,
               aej_ref, xa_ref):
    e = embp_ref[...]
    lw = linW_ref[...]
    pieces = []
    asrc, adst = [], []
    e_src = jnp.sum(e * aej_ref[...], axis=1, keepdims=True)
    e_dst = jnp.sum(e * aei_ref[...], axis=1, keepdims=True)
    for b in range(B):
        xl = jax.lax.dot_general(datap_ref[b], lw,
                                 (((1,), (1,)), ((), ())),
                                 preferred_element_type=jnp.float32)
        pieces.append(xl)
        asrc.append(jnp.sum(xl * aj_ref[...], axis=1, keepdims=True) + e_src)
        adst.append(jnp.sum(xl * ai_ref[...], axis=1, keepdims=True) + e_dst)
    zeros = jnp.zeros((BR, 8), jnp.float32)
    xa_ref[...] = jnp.concatenate(pieces + asrc + adst + [zeros], axis=1)


def _post_body(agg_ref, embp_ref, bias_ref, bn1g_ref, bn1b_ref, bog_ref,
               bob_ref, ow_ref, ob_ref, y_ref):
    e = embp_ref[...]
    rows = []
    for b in range(B):
        agg = agg_ref[:, b * D:(b + 1) * D] + bias_ref[...]
        o = jnp.maximum(bn1g_ref[...] * agg * C_BN + bn1b_ref[...], 0.0)
        h = o * e
        h = jnp.maximum(bog_ref[...] * h * C_BN + bob_ref[...], 0.0)
        yb = jnp.sum(h * ow_ref[...], axis=1) + ob_ref[0, 0]
        rows.append(yb[None, :])
    y_ref[...] = jnp.concatenate(rows, axis=0)


NTILE = 32
NPT = NPAD // NTILE
GRP = 8
NGRP = NPT // GRP
ROWS_G = GRP * KP


def _sc_mp_body(idx_hbm, xa_hbm, out_hbm, idxbuf, rows, accg, sem):
    wid = lax.axis_index("s") * 2 + lax.axis_index("c")
    io = lax.iota(jnp.int32, 16)
    fz = jnp.zeros((16,), jnp.float32)

    def group_body(g, _):
        base = wid * NPT + g * GRP
        pltpu.sync_copy(idx_hbm.at[pl.ds(base * KP, 96)], idxbuf.at[0])
        pltpu.sync_copy(idx_hbm.at[pl.ds(base * KP + 96, 96)], idxbuf.at[1])
        c0 = pltpu.async_copy(xa_hbm.at[idxbuf.at[0]],
                              rows.at[pl.ds(0, 96)], sem)
        c1 = pltpu.async_copy(xa_hbm.at[idxbuf.at[1]],
                              rows.at[pl.ds(96, 96)], sem)
        c0.wait()
        c1.wait()

        def node_body(n, _n):
            i = base + n
            sb = n * KP
            r = n // 4
            off = (n % 4) * KP
            iv0 = idxbuf[r, pl.ds(off, 16)]
            iv1 = idxbuf[r, pl.ds(off + 8, 16)]
            valid0 = iv0 != i
            valid1 = ((iv1 != i) | (io == 12)) & (io >= 8)
            w0, w1 = [], []
            for b in range(B):
                asrc0 = plsc.load_gather(
                    rows, [sb + io, jnp.full((16,), 4 * D + b, jnp.int32)])
                asrc1 = plsc.load_gather(
                    rows, [sb + 8 + io, jnp.full((16,), 4 * D + b, jnp.int32)])
                adv = plsc.load_gather(
                    rows, [jnp.full((16,), sb + K, jnp.int32),
                           jnp.full((16,), 4 * D + B + b, jnp.int32)])
                a0 = adv + asrc0
                a0 = jnp.where(a0 >= 0, a0, 0.2 * a0)
                a1 = adv + asrc1
                a1 = jnp.where(a1 >= 0, a1, 0.2 * a1)
                m = jnp.maximum(jnp.max(jnp.where(valid0, a0, NEG)),
                                jnp.max(jnp.where(valid1, a1, NEG)))
                e0 = jnp.where(valid0, jnp.exp(a0 - m), fz)
                e1 = jnp.where(valid1, jnp.exp(a1 - m), fz)
                den = jnp.broadcast_to(jnp.sum(e0) + jnp.sum(e1) + 1e-16,
                                       (16,))
                w0.append(e0 / den)
                w1.append(e1 / den)
            acc = [fz] * 16
            dn = lax.GatherDimensionNumbers(offset_dims=(),
                                            collapsed_slice_dims=(0,),
                                            start_index_map=(0,))
            for t in range(K + 1):
                lane = jnp.full((16, 1), t if t < 16 else t - 8, jnp.int32)
                wts = [lax.gather(w0[b] if t < 16 else w1[b], lane, dn, (1,),
                                  mode=lax.GatherScatterMode.PROMISE_IN_BOUNDS)
                       for b in range(B)]
                for c in range(16):
                    acc[c] = acc[c] + wts[c // 4] * rows[sb + t,
                                                         pl.ds(c * 16, 16)]
            for c in range(16):
                accg[n, pl.ds(c * 16, 16)] = acc[c]
            return _n

        lax.fori_loop(0, GRP, node_body, 0)
        pltpu.sync_copy(accg, out_hbm.at[pl.ds(base, GRP)])
        return _

    lax.fori_loop(0, NGRP, group_body, 0)


def _sc_mp(idx_flat, xa):
    mesh = plsc.VectorSubcoreMesh(core_axis_name="c", subcore_axis_name="s")
    return pl.kernel(
        _sc_mp_body,
        mesh=mesh,
        compiler_params=pltpu.CompilerParams(use_tc_tiling_on_sc=False,
                                             needs_layout_passes=False),
        out_type=jax.ShapeDtypeStruct((NPAD, B * D), jnp.float32),
        scratch_types=[
            pltpu.VMEM((2, 96), jnp.int32),
            pltpu.VMEM((ROWS_G, XA_W), jnp.float32),
            pltpu.VMEM((GRP, B * D), jnp.float32),
            pltpu.SemaphoreType.DMA,
        ],
    )(idx_flat, xa)


def _post(agg_all, embp, gnn_bias, bn1_gamma, bn1_beta, bn_out_gamma,
          bn_out_beta, out_W, out_b):
    yp = pl.pallas_call(
        _post_body,
        grid=(NBLK,),
        in_specs=[
            pl.BlockSpec((BR, B * D), lambda i: (i, 0)),
            pl.BlockSpec((BR, D), lambda i: (i, 0)),
            pl.BlockSpec((1, D), lambda i: (0, 0)),
            pl.BlockSpec((1, D), lambda i: (0, 0)),
            pl.BlockSpec((1, D), lambda i: (0, 0)),
            pl.BlockSpec((1, D), lambda i: (0, 0)),
            pl.BlockSpec((1, D), lambda i: (0, 0)),
            pl.BlockSpec((1, D), lambda i: (0, 0)),
            pl.BlockSpec((1, 1), lambda i: (0, 0)),
        ],
        out_specs=pl.BlockSpec((B, BR), lambda i: (0, i)),
        out_shape=jax.ShapeDtypeStruct((B, NPAD), jnp.float32),
    )(agg_all, embp, gnn_bias[None, :], bn1_gamma[None, :],
      bn1_beta[None, :], bn_out_gamma[None, :], bn_out_beta[None, :],
      out_W, out_b[:, None])
    return yp[:, :N]


def kernel(data, emb, lin_W, att_i, att_j, att_em_i, att_em_j, gnn_bias,
           bn1_gamma, bn1_beta, bn_out_gamma, bn_out_beta, out_W, out_b):
    f32 = jnp.float32
    embp = jnp.pad(emb, ((0, NPAD - N), (0, 0)))
    nw = embp / jnp.sqrt(jnp.sum(embp * embp, axis=1, keepdims=True))
    nw = jnp.where(jnp.isfinite(nw), nw, 0.0)

    topk = pl.pallas_call(
        _topk_body,
        grid=(NBLK,),
        in_specs=[
            pl.BlockSpec((NPAD, D), lambda i: (0, 0)),
            pl.BlockSpec((BR, D), lambda i: (i, 0)),
        ],
        out_specs=pl.BlockSpec((BR, KP), lambda i: (i, 0)),
        out_shape=jax.ShapeDtypeStruct((NPAD, KP), jnp.int32),
    )(nw, nw)

    datap = jnp.pad(data, ((0, 0), (0, NPAD - N), (0, 0)))
    xa = pl.pallas_call(
        _prep_body,
        grid=(NBLK,),
        in_specs=[
            pl.BlockSpec((B, BR, data.shape[-1]), lambda i: (0, i, 0)),
            pl.BlockSpec((BR, D), lambda i: (i, 0)),
            pl.BlockSpec((D, data.shape[-1]), lambda i: (0, 0)),
            pl.BlockSpec((1, D), lambda i: (0, 0)),
            pl.BlockSpec((1, D), lambda i: (0, 0)),
            pl.BlockSpec((1, D), lambda i: (0, 0)),
            pl.BlockSpec((1, D), lambda i: (0, 0)),
        ],
        out_specs=pl.BlockSpec((BR, XA_W), lambda i: (i, 0)),
        out_shape=jax.ShapeDtypeStruct((NPAD, XA_W), f32),
    )(datap, embp, lin_W, att_i[0], att_j[0], att_em_i[0], att_em_j[0])

    agg_all = _sc_mp(topk.reshape(-1), xa)
    return _post(agg_all, embp, gnn_bias, bn1_gamma, bn1_beta,
                 bn_out_gamma, bn_out_beta, out_W, out_b)


# speedup vs baseline: 3.8625x; 1.0159x over previous
import functools

import jax
import jax.numpy as jnp
import numpy as np
from jax import lax
from jax.experimental import pallas as pl
from jax.experimental.pallas import tpu as pltpu
from jax.experimental.pallas import tpu_sc as plsc

N = 10000
NPAD = 10240
D = 64
B = 4
K = 20
KP = 24
BR = 640
NBLK = NPAD // BR
XA_W = 272
C_BN = float(1.0 / np.sqrt(1.0 + 1e-5))
NEG = float(-3.0e38)


def _topk_body(nw_all_ref, nw_blk_ref, out_ref):
    s = jax.lax.dot_general(nw_blk_ref[...], nw_all_ref[...],
                            (((1,), (1,)), ((), ())),
                            preferred_element_type=jnp.float32)
    col = jax.lax.broadcasted_iota(jnp.int32, (BR, NPAD), 1)
    s = jnp.where(col < N, s, NEG)
    sels = []
    m = jnp.max(s, axis=1, keepdims=True)
    for t in range(K):
        eq = s == m
        cand = jnp.where(eq, col, NPAD)
        sel = jnp.min(cand, axis=1, keepdims=True)
        s = jnp.where(eq, NEG, s)
        if t < K - 1:
            m = jnp.max(s, axis=1, keepdims=True)
        sels.append(sel)
    pid = pl.program_id(0)
    row = pid * BR + jax.lax.broadcasted_iota(jnp.int32, (BR, 1), 0)
    out_ref[...] = jnp.concatenate(sels + [row] * (KP - K), axis=1)


def _prep_body(datap_ref, embp_ref, linW_ref, ai_ref, aj_ref, aei_ref,
               aej_ref, xa_ref):
    e = embp_ref[...]
    lw = linW_ref[...]
    pieces = []
    asrc, adst = [], []
    e_src = jnp.sum(e * aej_ref[...], axis=1, keepdims=True)
    e_dst = jnp.sum(e * aei_ref[...], axis=1, keepdims=True)
    for b in range(B):
        xl = jax.lax.dot_general(datap_ref[b], lw,
                                 (((1,), (1,)), ((), ())),
                                 preferred_element_type=jnp.float32)
        pieces.append(xl)
        asrc.append(jnp.sum(xl * aj_ref[...], axis=1, keepdims=True) + e_src)
        adst.append(jnp.sum(xl * ai_ref[...], axis=1, keepdims=True) + e_dst)
    zeros = jnp.zeros((BR, 8), jnp.float32)
    xa_ref[...] = jnp.concatenate(pieces + asrc + adst + [zeros], axis=1)


def _post_body(agg_ref, embp_ref, bias_ref, bn1g_ref, bn1b_ref, bog_ref,
               bob_ref, ow_ref, ob_ref, y_ref):
    e = embp_ref[...]
    rows = []
    for b in range(B):
        agg = agg_ref[:, b * D:(b + 1) * D] + bias_ref[...]
        o = jnp.maximum(bn1g_ref[...] * agg * C_BN + bn1b_ref[...], 0.0)
        h = o * e
        h = jnp.maximum(bog_ref[...] * h * C_BN + bob_ref[...], 0.0)
        yb = jnp.sum(h * ow_ref[...], axis=1) + ob_ref[0, 0]
        rows.append(yb[None, :])
    y_ref[...] = jnp.concatenate(rows, axis=0)


NTILE = 32
NPT = NPAD // NTILE
GRP = 8
NGRP = NPT // GRP
ROWS_G = GRP * KP


def _sc_mp_body(idx_hbm, xa_hbm, out_hbm, idxbuf, rows, accg, sem):
    wid = lax.axis_index("s") * 2 + lax.axis_index("c")
    io = lax.iota(jnp.int32, 16)
    fz = jnp.zeros((16,), jnp.float32)

    def group_body(g, _):
        base = wid * NPT + g * GRP
        pltpu.sync_copy(idx_hbm.at[pl.ds(base * KP, 96)], idxbuf.at[0])
        pltpu.sync_copy(idx_hbm.at[pl.ds(base * KP + 96, 96)], idxbuf.at[1])
        c0 = pltpu.async_copy(xa_hbm.at[idxbuf.at[0]],
                              rows.at[pl.ds(0, 96)], sem)
        c1 = pltpu.async_copy(xa_hbm.at[idxbuf.at[1]],
                              rows.at[pl.ds(96, 96)], sem)
        c0.wait()
        c1.wait()

        def node_body(n, _n):
            i = base + n
            sb = n * KP
            r = n // 4
            off = (n % 4) * KP
            iv0 = idxbuf[r, pl.ds(off, 16)]
            iv1 = idxbuf[r, pl.ds(off + 8, 16)]
            valid0 = iv0 != i
            valid1 = ((iv1 != i) | (io == 12)) & (io >= 8)
            w0, w1 = [], []
            for b in range(B):
                asrc0 = plsc.load_gather(
                    rows, [sb + io, jnp.full((16,), 4 * D + b, jnp.int32)])
                asrc1 = plsc.load_gather(
                    rows, [sb + 8 + io, jnp.full((16,), 4 * D + b, jnp.int32)])
                adv = plsc.load_gather(
                    rows, [jnp.full((16,), sb + K, jnp.int32),
                           jnp.full((16,), 4 * D + B + b, jnp.int32)])
                a0 = adv + asrc0
                a0 = jnp.where(a0 >= 0, a0, 0.2 * a0)
                a1 = adv + asrc1
                a1 = jnp.where(a1 >= 0, a1, 0.2 * a1)
                m = jnp.maximum(jnp.max(jnp.where(valid0, a0, NEG)),
                                jnp.max(jnp.where(valid1, a1, NEG)))
                e0 = jnp.where(valid0, jnp.exp(a0 - m), fz)
                e1 = jnp.where(valid1, jnp.exp(a1 - m), fz)
                den = jnp.broadcast_to(jnp.sum(e0) + jnp.sum(e1) + 1e-16,
                                       (16,))
                w0.append(e0 / den)
                w1.append(e1 / den)
            acc = [fz] * 16
            dn = lax.GatherDimensionNumbers(offset_dims=(),
                                            collapsed_slice_dims=(0,),
                                            start_index_map=(0,))
            for t in range(K + 1):
                lane = jnp.full((16, 1), t if t < 16 else t - 8, jnp.int32)
                wts = [lax.gather(w0[b] if t < 16 else w1[b], lane, dn, (1,),
                                  mode=lax.GatherScatterMode.PROMISE_IN_BOUNDS)
                       for b in range(B)]
                for c in range(16):
                    acc[c] = acc[c] + wts[c // 4] * rows[sb + t,
                                                         pl.ds(c * 16, 16)]
            for c in range(16):
                accg[n, pl.ds(c * 16, 16)] = acc[c]
            return _n

        lax.fori_loop(0, GRP, node_body, 0)
        pltpu.sync_copy(accg, out_hbm.at[pl.ds(base, GRP)])
        return _

    lax.fori_loop(0, NGRP, group_body, 0)


def _sc_mp(idx_flat, xa):
    mesh = plsc.VectorSubcoreMesh(core_axis_name="c", subcore_axis_name="s")
    return pl.kernel(
        _sc_mp_body,
        mesh=mesh,
        compiler_params=pltpu.CompilerParams(use_tc_tiling_on_sc=False,
                                             needs_layout_passes=False),
        out_type=jax.ShapeDtypeStruct((NPAD, B * D), jnp.float32),
        scratch_types=[
            pltpu.VMEM((2, 96), jnp.int32),
            pltpu.VMEM((ROWS_G, XA_W), jnp.float32),
            pltpu.VMEM((GRP, B * D), jnp.float32),
            pltpu.SemaphoreType.DMA,
        ],
    )(idx_flat, xa)


def _post(agg_all, embp, gnn_bias, bn1_gamma, bn1_beta, bn_out_gamma,
          bn_out_beta, out_W, out_b):
    yp = pl.pallas_call(
        _post_body,
        grid=(NBLK,),
        in_specs=[
            pl.BlockSpec((BR, B * D), lambda i: (i, 0)),
            pl.BlockSpec((BR, D), lambda i: (i, 0)),
            pl.BlockSpec((1, D), lambda i: (0, 0)),
            pl.BlockSpec((1, D), lambda i: (0, 0)),
            pl.BlockSpec((1, D), lambda i: (0, 0)),
            pl.BlockSpec((1, D), lambda i: (0, 0)),
            pl.BlockSpec((1, D), lambda i: (0, 0)),
            pl.BlockSpec((1, D), lambda i: (0, 0)),
            pl.BlockSpec((1, 1), lambda i: (0, 0)),
        ],
        out_specs=pl.BlockSpec((B, BR), lambda i: (0, i)),
        out_shape=jax.ShapeDtypeStruct((B, NPAD), jnp.float32),
    )(agg_all, embp, gnn_bias[None, :], bn1_gamma[None, :],
      bn1_beta[None, :], bn_out_gamma[None, :], bn_out_beta[None, :],
      out_W, out_b[:, None])
    return yp[:, :N]


def kernel(data, emb, lin_W, att_i, att_j, att_em_i, att_em_j, gnn_bias,
           bn1_gamma, bn1_beta, bn_out_gamma, bn_out_beta, out_W, out_b):
    f32 = jnp.float32
    embp = jnp.pad(emb, ((0, NPAD - N), (0, 0)))
    nw = embp / jnp.sqrt(jnp.sum(embp * embp, axis=1, keepdims=True))
    nw = jnp.where(jnp.isfinite(nw), nw, 0.0)

    topk = pl.pallas_call(
        _topk_body,
        grid=(NBLK,),
        in_specs=[
            pl.BlockSpec((NPAD, D), lambda i: (0, 0)),
            pl.BlockSpec((BR, D), lambda i: (i, 0)),
        ],
        out_specs=pl.BlockSpec((BR, KP), lambda i: (i, 0)),
        out_shape=jax.ShapeDtypeStruct((NPAD, KP), jnp.int32),
    )(nw, nw)

    datap = jnp.pad(data, ((0, 0), (0, NPAD - N), (0, 0)))
    xa = pl.pallas_call(
        _prep_body,
        grid=(NBLK,),
        in_specs=[
            pl.BlockSpec((B, BR, data.shape[-1]), lambda i: (0, i, 0)),
            pl.BlockSpec((BR, D), lambda i: (i, 0)),
            pl.BlockSpec((D, data.shape[-1]), lambda i: (0, 0)),
            pl.BlockSpec((1, D), lambda i: (0, 0)),
            pl.BlockSpec((1, D), lambda i: (0, 0)),
            pl.BlockSpec((1, D), lambda i: (0, 0)),
            pl.BlockSpec((1, D), lambda i: (0, 0)),
        ],
        out_specs=pl.BlockSpec((BR, XA_W), lambda i: (i, 0)),
        out_shape=jax.ShapeDtypeStruct((NPAD, XA_W), f32),
    )(datap, embp, lin_W, att_i[0], att_j[0], att_em_i[0], att_em_j[0])

    agg_all = _sc_mp(topk.reshape(-1), xa)
    return _post(agg_all, embp, gnn_bias, bn1_gamma, bn1_beta,
                 bn_out_gamma, bn_out_beta, out_W, out_b)
